# baseline (speedup 1.0000x reference)
import functools

import jax
import jax.numpy as jnp
from jax import lax
from jax.experimental import pallas as pl
from jax.experimental.pallas import tpu as pltpu
from jax.experimental.pallas import tpu_sc as plsc

_NUM_BATCHES = 4
_RES = 128
_OCC_THRE = 0.01
_EMA_DECAY = 0.95
_NUM_PTS = 1048576

_NC = 2
_NS = 16
_NW = _NC * _NS
_L = 16

_GRID = _NUM_BATCHES * _RES * _RES * _RES
_NBUK = 128
_V = _GRID // _NBUK
_BUK_PER_W = _NBUK // _NW

_P = _NUM_PTS // _NW
_CP = 2048
_CS = 4096

_mesh = plsc.VectorSubcoreMesh(
    core_axis_name="c", subcore_axis_name="s", num_cores=_NC, num_subcores=_NS
)
_sc_params = pltpu.CompilerParams(needs_layout_passes=False)


def _wid():
    return lax.axis_index("s") * _NC + lax.axis_index("c")


def _iota():
    return lax.iota(jnp.int32, _L)


@functools.partial(
    pl.kernel,
    out_type=jax.ShapeDtypeStruct((_NUM_PTS,), jnp.int32),
    mesh=_mesh,
    scratch_types=[
        pltpu.VMEM((_CP * 3,), jnp.float32),
        pltpu.VMEM((_CP,), jnp.int32),
        pltpu.VMEM((_CP,), jnp.int32),
    ],
    compiler_params=_sc_params,
)
def _phase_a(pts_hbm, bidx_hbm, idx_hbm, pts_st, bidx_st, out_st):
    base = _wid() * _P
    lanes = _iota()

    def chunk(c, _):
        off = base + c * _CP
        pltpu.sync_copy(pts_hbm.at[pl.ds(off * 3, _CP * 3)], pts_st)
        pltpu.sync_copy(bidx_hbm.at[pl.ds(off, _CP)], bidx_st)

        def vec(k, _):
            g = lanes * 3 + k * (3 * _L)
            x = plsc.load_gather(pts_st, [g])
            y = plsc.load_gather(pts_st, [g + 1])
            z = plsc.load_gather(pts_st, [g + 2])
            b = bidx_st[pl.ds(k * _L, _L)]
            xi = jnp.clip((x * float(_RES)).astype(jnp.int32), 0, _RES - 1)
            yi = jnp.clip((y * float(_RES)).astype(jnp.int32), 0, _RES - 1)
            zi = jnp.clip((z * float(_RES)).astype(jnp.int32), 0, _RES - 1)
            flat = (b << 21) | (xi << 14) | (yi << 7) | zi
            out_st[pl.ds(k * _L, _L)] = flat
            return 0

        lax.fori_loop(0, _CP // _L, vec, 0)
        pltpu.sync_copy(out_st, idx_hbm.at[pl.ds(off, _CP)])
        return 0

    lax.fori_loop(0, _P // _CP, chunk, 0)


@functools.partial(
    pl.kernel,
    out_type=jax.ShapeDtypeStruct((_GRID,), jnp.float32),
    mesh=_mesh,
    scratch_types=[
        pltpu.VMEM((_V,), jnp.float32),
        pltpu.VMEM((_CS,), jnp.int32),
        pltpu.VMEM((_CS,), jnp.float32),
    ],
    compiler_params=_sc_params,
)
def _phase_b(occ_hbm, idx_hbm, val_hbm, new_hbm, acc, idx_st, val_st):
    w = _wid()

    for r in range(_BUK_PER_W):
        bucket = w * _BUK_PER_W + r
        gbase = bucket * _V
        pltpu.sync_copy(occ_hbm.at[pl.ds(gbase, _V)], acc)

        def decay(k, _):
            acc[pl.ds(k * _L, _L)] = acc[pl.ds(k * _L, _L)] * _EMA_DECAY
            return 0

        lax.fori_loop(0, _V // _L, decay, 0)

        def chunk(c, _):
            pltpu.sync_copy(idx_hbm.at[pl.ds(c * _CS, _CS)], idx_st)
            pltpu.sync_copy(val_hbm.at[pl.ds(c * _CS, _CS)], val_st)

            def vec(k, _):
                vi = idx_st[pl.ds(k * _L, _L)]
                vv = val_st[pl.ds(k * _L, _L)]
                m0 = lax.shift_right_logical(vi, 16) == bucket
                lo = vi & (_V - 1)

                def retry(m):
                    g = plsc.load_gather(acc, [lo], mask=m)
                    plsc.store_scatter(acc, [lo], jnp.maximum(g, vv), mask=m)
                    g2 = plsc.load_gather(acc, [lo], mask=m)
                    return m & (g2 < vv)

                lax.while_loop(lambda m: jnp.any(m), retry, m0)
                return 0

            lax.fori_loop(0, _CS // _L, vec, 0)
            return 0

        lax.fori_loop(0, _NUM_PTS // _CS, chunk, 0)
        pltpu.sync_copy(acc, new_hbm.at[pl.ds(gbase, _V)])


def _thr_body(x_ref, o_ref):
    o_ref[...] = x_ref[...] > _OCC_THRE


_ROWS = _GRID // 128


def _phase_c(new2d):
    return pl.pallas_call(
        _thr_body,
        out_shape=jax.ShapeDtypeStruct((_ROWS, 128), jnp.bool_),
        grid=(32,),
        in_specs=[pl.BlockSpec((_ROWS // 32, 128), lambda i: (i, 0))],
        out_specs=pl.BlockSpec((_ROWS // 32, 128), lambda i: (i, 0)),
    )(new2d)


def kernel(occ_val_grid, pts, bidx, val):
    pts_flat = pts.reshape(-1)
    occ_flat = occ_val_grid.reshape(-1)
    idxs = _phase_a(pts_flat, bidx)
    new_flat = _phase_b(occ_flat, idxs, val)
    occ_bool = _phase_c(new_flat.reshape(_ROWS, 128))
    shape = occ_val_grid.shape
    return new_flat.reshape(shape), occ_bool.reshape(shape)


# speedup vs baseline: 5.7879x; 5.7879x over previous
import functools

import jax
import jax.numpy as jnp
from jax import lax
from jax.experimental import pallas as pl
from jax.experimental.pallas import tpu as pltpu
from jax.experimental.pallas import tpu_sc as plsc

_NUM_BATCHES = 4
_RES = 128
_OCC_THRE = 0.01
_EMA_DECAY = 0.95
_NUM_PTS = 1048576

_NC = 2
_NS = 16
_NW = _NC * _NS
_L = 16

_GRID = _NUM_BATCHES * _RES * _RES * _RES
_NBUK = 128
_V = _GRID // _NBUK
_BUK_PER_W = _NBUK // _NW

_P = _NUM_PTS // _NW
_CP = 2048
_CH = 512
_WIN = _CH + 16

_BINNED_N = 2 * _NUM_PTS + 4096

_mesh = plsc.VectorSubcoreMesh(
    core_axis_name="c", subcore_axis_name="s", num_cores=_NC, num_subcores=_NS
)
_sc_params = pltpu.CompilerParams(needs_layout_passes=False)


def _wid():
    return lax.axis_index("s") * _NC + lax.axis_index("c")


def _iota():
    return lax.iota(jnp.int32, _L)


@functools.partial(
    pl.kernel,
    out_type=(
        jax.ShapeDtypeStruct((_BINNED_N,), jnp.int32),
        jax.ShapeDtypeStruct((_NW * _NBUK,), jnp.int32),
    ),
    mesh=_mesh,
    scratch_types=[
        pltpu.VMEM((_P,), jnp.int32),
        pltpu.VMEM((2 * _P,), jnp.int32),
        pltpu.VMEM((_CP * 3,), jnp.float32),
        pltpu.VMEM((_CP,), jnp.int32),
        pltpu.VMEM((_CP,), jnp.float32),
        pltpu.VMEM((_NBUK * _L,), jnp.int32),
        pltpu.VMEM((_NBUK * _L,), jnp.int32),
        pltpu.VMEM((_NBUK,), jnp.int32),
    ],
    compiler_params=_sc_params,
)
def _phase1(pts_hbm, bidx_hbm, val_hbm, binned_hbm, tot_hbm,
            idxbuf, binned_st, pts_st, bidx_st, val_st, cnt16, off16, tot_st):
    w = _wid()
    base = w * _P
    lanes = _iota()
    zeros = jnp.zeros((_L,), jnp.int32)

    def zcnt(k, _):
        cnt16[pl.ds(k * _L, _L)] = zeros
        return 0

    lax.fori_loop(0, _NBUK * _L // _L, zcnt, 0)

    def chunk_a(c, _):
        off = base + c * _CP
        pltpu.sync_copy(pts_hbm.at[pl.ds(off * 3, _CP * 3)], pts_st)
        pltpu.sync_copy(bidx_hbm.at[pl.ds(off, _CP)], bidx_st)

        def vec(k, _):
            g = lanes * 3 + k * (3 * _L)
            x = plsc.load_gather(pts_st, [g])
            y = plsc.load_gather(pts_st, [g + 1])
            z = plsc.load_gather(pts_st, [g + 2])
            b = bidx_st[pl.ds(k * _L, _L)]
            xi = jnp.clip((x * float(_RES)).astype(jnp.int32), 0, _RES - 1)
            yi = jnp.clip((y * float(_RES)).astype(jnp.int32), 0, _RES - 1)
            zi = jnp.clip((z * float(_RES)).astype(jnp.int32), 0, _RES - 1)
            flat = (b << 21) | (xi << 14) | (yi << 7) | zi
            idxbuf[pl.ds(c * _CP + k * _L, _L)] = flat
            cidx = lax.shift_right_logical(flat, 16) * _L + lanes
            cur = plsc.load_gather(cnt16, [cidx])
            plsc.store_scatter(cnt16, [cidx], cur + 1)
            return 0

        lax.fori_loop(0, _CP // _L, vec, 0)
        return 0

    lax.fori_loop(0, _P // _CP, chunk_a, 0)

    def pref(b, c):
        v = cnt16[pl.ds(b * _L, _L)]
        cs = plsc.cumsum(v)
        t = jnp.sum(v)
        off16[pl.ds(b * _L, _L)] = (cs - v) + c
        plsc.store_scatter(tot_st, [zeros + b], zeros + t, mask=lanes == 0)
        return c + t

    lax.fori_loop(0, _NBUK, pref, 0)

    def chunk_b(c, _):
        off = base + c * _CP
        pltpu.sync_copy(val_hbm.at[pl.ds(off, _CP)], val_st)

        def vec(k, _):
            flat = idxbuf[pl.ds(c * _CP + k * _L, _L)]
            vv = val_st[pl.ds(k * _L, _L)]
            cidx = lax.shift_right_logical(flat, 16) * _L + lanes
            p = plsc.load_gather(off16, [cidx])
            plsc.store_scatter(off16, [cidx], p + 1)
            plsc.store_scatter(binned_st, [p * 2], flat)
            plsc.store_scatter(binned_st, [p * 2 + 1], plsc.bitcast(vv, jnp.int32))
            return 0

        lax.fori_loop(0, _CP // _L, vec, 0)
        return 0

    lax.fori_loop(0, _P // _CP, chunk_b, 0)

    pltpu.sync_copy(binned_st, binned_hbm.at[pl.ds(w * (2 * _P), 2 * _P)])
    pltpu.sync_copy(tot_st, tot_hbm.at[pl.ds(w * _NBUK, _NBUK)])


@functools.partial(
    pl.kernel,
    out_type=jax.ShapeDtypeStruct((_GRID,), jnp.float32),
    mesh=_mesh,
    scratch_types=[
        pltpu.VMEM((_V,), jnp.float32),
        pltpu.VMEM((_NW * _NBUK,), jnp.int32),
        pltpu.VMEM((2 * _WIN,), jnp.int32),
    ],
    compiler_params=_sc_params,
)
def _phase2(occ_hbm, binned_hbm, tot_hbm, new_hbm, acc, tot_st, pair_st):
    w = _wid()
    lanes = _iota()
    pltpu.sync_copy(tot_hbm, tot_st)

    for r in range(_BUK_PER_W):
        bucket = w * _BUK_PER_W + r
        gbase = bucket * _V
        pltpu.sync_copy(occ_hbm.at[pl.ds(gbase, _V)], acc)

        def decay(k, _):
            acc[pl.ds(k * _L, _L)] = acc[pl.ds(k * _L, _L)] * _EMA_DECAY
            return 0

        lax.fori_loop(0, _V // _L, decay, 0)

        def seg(sw, _):
            def row(g, carry):
                pre, ln = carry
                v = tot_st[pl.ds(sw * _NBUK + g * _L, _L)]
                gl = g * _L + lanes
                pre = pre + jnp.sum(jnp.where(gl < bucket, v, 0))
                ln = ln + jnp.sum(jnp.where(gl == bucket, v, 0))
                return pre, ln

            pre, ln = lax.fori_loop(
                0, _NBUK // _L, row, (jnp.int32(0), jnp.int32(0)))
            p0 = sw * _P + pre
            nch = lax.shift_right_logical(ln + _CH - 1, _CH.bit_length() - 1)

            def chunk(j, _):
                pj = p0 + j * _CH
                a0 = pl.multiple_of((pj * 2) & ~15, 8)
                shp = pj & 7
                pltpu.sync_copy(binned_hbm.at[pl.ds(a0, 2 * _WIN)], pair_st)

                def vec(k, _):
                    wpos = k * _L + lanes
                    vi = plsc.load_gather(pair_st, [wpos * 2])
                    vv = plsc.bitcast(
                        plsc.load_gather(pair_st, [wpos * 2 + 1]), jnp.float32)
                    q = wpos - shp + j * _CH
                    m0 = (wpos >= shp) & (wpos < shp + _CH) & (q < ln)
                    lo = vi & (_V - 1)

                    def retry(m):
                        g = plsc.load_gather(acc, [lo], mask=m)
                        plsc.store_scatter(acc, [lo], jnp.maximum(g, vv), mask=m)
                        g2 = plsc.load_gather(acc, [lo], mask=m)
                        return m & (g2 < vv)

                    lax.while_loop(lambda m: jnp.any(m), retry, m0)
                    return 0

                lax.fori_loop(0, _WIN // _L, vec, 0)
                return 0

            lax.fori_loop(0, nch, chunk, 0)
            return 0

        lax.fori_loop(0, _NW, seg, 0)
        pltpu.sync_copy(acc, new_hbm.at[pl.ds(gbase, _V)])


def _thr_body(x_ref, o_ref):
    o_ref[...] = x_ref[...] > _OCC_THRE


_ROWS = _GRID // 128


def _phase3(new2d):
    return pl.pallas_call(
        _thr_body,
        out_shape=jax.ShapeDtypeStruct((_ROWS, 128), jnp.bool_),
        grid=(32,),
        in_specs=[pl.BlockSpec((_ROWS // 32, 128), lambda i: (i, 0))],
        out_specs=pl.BlockSpec((_ROWS // 32, 128), lambda i: (i, 0)),
    )(new2d)


def kernel(occ_val_grid, pts, bidx, val):
    pts_flat = pts.reshape(-1)
    occ_flat = occ_val_grid.reshape(-1)
    binned, totals = _phase1(pts_flat, bidx, val)
    new_flat = _phase2(occ_flat, binned, totals)
    occ_bool = _phase3(new_flat.reshape(_ROWS, 128))
    shape = occ_val_grid.shape
    return new_flat.reshape(shape), occ_bool.reshape(shape)


# speedup vs baseline: 27.7434x; 4.7933x over previous
import functools

import jax
import jax.numpy as jnp
from jax import lax
from jax.experimental import pallas as pl
from jax.experimental.pallas import tpu as pltpu
from jax.experimental.pallas import tpu_sc as plsc

_NUM_BATCHES = 4
_RES = 128
_OCC_THRE = 0.01
_EMA_DECAY = 0.95
_NUM_PTS = 1048576

_NC = 2
_NS = 16
_NW = _NC * _NS
_L = 16

_GRID = _NUM_BATCHES * _RES * _RES * _RES
_NBUK = 128
_V = _GRID // _NBUK
_BUK_PER_W = _NBUK // _NW

_P = _NUM_PTS // _NW
_CP = 2048
_CH = 512
_WIN = _CH + 16
_WINW = 2 * _WIN

_BINNED_N = 2 * _NUM_PTS + 4096

_mesh = plsc.VectorSubcoreMesh(
    core_axis_name="c", subcore_axis_name="s", num_cores=_NC, num_subcores=_NS
)
_sc_params = pltpu.CompilerParams(needs_layout_passes=False)


def _wid():
    return lax.axis_index("s") * _NC + lax.axis_index("c")


def _iota():
    return lax.iota(jnp.int32, _L)


@functools.partial(
    pl.kernel,
    out_type=(
        jax.ShapeDtypeStruct((_BINNED_N,), jnp.int32),
        jax.ShapeDtypeStruct((_NW * _NBUK,), jnp.int32),
    ),
    mesh=_mesh,
    scratch_types=[
        pltpu.VMEM((_P,), jnp.int32),
        pltpu.VMEM((2 * _P,), jnp.int32),
        pltpu.VMEM((3 * _CP,), jnp.float32),
        pltpu.VMEM((_CP,), jnp.int32),
        pltpu.VMEM((_CP,), jnp.float32),
        pltpu.VMEM((_NBUK * _L,), jnp.int32),
        pltpu.VMEM((_NBUK * _L,), jnp.int32),
        pltpu.VMEM((_NBUK,), jnp.int32),
    ],
    compiler_params=_sc_params,
)
def _phase1(pts_hbm, bidx_hbm, val_hbm, binned_hbm, tot_hbm,
            idxbuf, binned_st, pts_st, bidx_st, val_st, cnt16, off16, tot_st):
    w = _wid()
    base = w * _P
    lanes = _iota()
    zeros = jnp.zeros((_L,), jnp.int32)

    def zcnt(k, _):
        cnt16[pl.ds(k * _L, _L)] = zeros
        return 0

    lax.fori_loop(0, _NBUK * _L // _L, zcnt, 0)

    def chunk_a(c, _):
        off = base + c * _CP
        pltpu.sync_copy(pts_hbm.at[pl.ds(off, _CP)], pts_st.at[pl.ds(0, _CP)])
        pltpu.sync_copy(pts_hbm.at[pl.ds(_NUM_PTS + off, _CP)],
                        pts_st.at[pl.ds(_CP, _CP)])
        pltpu.sync_copy(pts_hbm.at[pl.ds(2 * _NUM_PTS + off, _CP)],
                        pts_st.at[pl.ds(2 * _CP, _CP)])
        pltpu.sync_copy(bidx_hbm.at[pl.ds(off, _CP)], bidx_st)

        def vec(k, _):
            x = pts_st[pl.ds(k * _L, _L)]
            y = pts_st[pl.ds(_CP + k * _L, _L)]
            z = pts_st[pl.ds(2 * _CP + k * _L, _L)]
            b = bidx_st[pl.ds(k * _L, _L)]
            xi = jnp.clip((x * float(_RES)).astype(jnp.int32), 0, _RES - 1)
            yi = jnp.clip((y * float(_RES)).astype(jnp.int32), 0, _RES - 1)
            zi = jnp.clip((z * float(_RES)).astype(jnp.int32), 0, _RES - 1)
            flat = (b << 21) | (xi << 14) | (yi << 7) | zi
            idxbuf[pl.ds(c * _CP + k * _L, _L)] = flat
            cidx = lax.shift_right_logical(flat, 16) * _L + lanes
            cur = plsc.load_gather(cnt16, [cidx])
            plsc.store_scatter(cnt16, [cidx], cur + 1)
            return 0

        lax.fori_loop(0, _CP // _L, vec, 0)
        return 0

    lax.fori_loop(0, _P // _CP, chunk_a, 0)

    def pref(b, c):
        v = cnt16[pl.ds(b * _L, _L)]
        cs = plsc.cumsum(v)
        t = jnp.sum(v)
        off16[pl.ds(b * _L, _L)] = (cs - v) + c
        plsc.store_scatter(tot_st, [zeros + b], zeros + t, mask=lanes == 0)
        return c + t

    lax.fori_loop(0, _NBUK, pref, 0)

    def chunk_b(c, _):
        off = base + c * _CP
        pltpu.sync_copy(val_hbm.at[pl.ds(off, _CP)], val_st)

        def vec(k, _):
            flat = idxbuf[pl.ds(c * _CP + k * _L, _L)]
            vv = val_st[pl.ds(k * _L, _L)]
            cidx = lax.shift_right_logical(flat, 16) * _L + lanes
            p = plsc.load_gather(off16, [cidx])
            plsc.store_scatter(off16, [cidx], p + 1)
            plsc.store_scatter(binned_st, [p * 2], flat)
            plsc.store_scatter(binned_st, [p * 2 + 1], plsc.bitcast(vv, jnp.int32))
            return 0

        lax.fori_loop(0, _CP // _L, vec, 0)
        return 0

    lax.fori_loop(0, _P // _CP, chunk_b, 0)

    pltpu.sync_copy(binned_st, binned_hbm.at[pl.ds(w * (2 * _P), 2 * _P)])
    pltpu.sync_copy(tot_st, tot_hbm.at[pl.ds(w * _NBUK, _NBUK)])


@functools.partial(
    pl.kernel,
    out_type=jax.ShapeDtypeStruct((_GRID,), jnp.float32),
    mesh=_mesh,
    scratch_types=[
        pltpu.VMEM((_V,), jnp.float32),
        pltpu.VMEM((_NW * _NBUK + _L,), jnp.int32),
        pltpu.VMEM((_NW * _WINW,), jnp.int32),
        pltpu.VMEM((_WINW,), jnp.int32),
        pltpu.VMEM((4 * _NW + _L,), jnp.int32),
        pltpu.SemaphoreType.DMA,
    ],
    compiler_params=_sc_params,
)
def _phase2(occ_hbm, binned_hbm, tot_hbm, new_hbm,
            acc, tot_st, seg_st, pair_st, par_st, sem):
    w = _wid()
    lanes = _iota()
    pltpu.sync_copy(tot_hbm, tot_st.at[pl.ds(0, _NW * _NBUK)])

    def accb(bp, c01):
        c0, c1 = c01
        c0 = c0 + plsc.load_gather(tot_st, [lanes * _NBUK + bp])
        c1 = c1 + plsc.load_gather(tot_st, [(lanes + _L) * _NBUK + bp])
        return c0, c1

    zeros = jnp.zeros((_L,), jnp.int32)
    pre0, pre1 = lax.fori_loop(0, w * _BUK_PER_W, accb, (zeros, zeros))

    def bucket_body(r, pre01):
        pre0, pre1 = pre01
        bucket = w * _BUK_PER_W + r
        gbase = bucket * _V

        ln0 = plsc.load_gather(tot_st, [lanes * _NBUK + bucket])
        ln1 = plsc.load_gather(tot_st, [(lanes + _L) * _NBUK + bucket])
        par_st[pl.ds(0, _L)] = lanes * _P + pre0
        par_st[pl.ds(_L, _L)] = (lanes + _L) * _P + pre1
        par_st[pl.ds(2 * _L, _L)] = ln0
        par_st[pl.ds(3 * _L, _L)] = ln1

        def fire(i, _):
            p0 = par_st[pl.ds(i, _L)][0]
            a0 = pl.multiple_of((p0 * 2) & ~15, 8)
            pltpu.async_copy(
                binned_hbm.at[pl.ds(a0, _WINW)],
                seg_st.at[pl.ds(i * _WINW, _WINW)], sem)
            return 0

        lax.fori_loop(0, _NW, fire, 0)

        pltpu.sync_copy(occ_hbm.at[pl.ds(gbase, _V)], acc)

        def decay(k, _):
            for u in range(4):
                o = k * (4 * _L) + u * _L
                acc[pl.ds(o, _L)] = acc[pl.ds(o, _L)] * _EMA_DECAY
            return 0

        lax.fori_loop(0, _V // (4 * _L), decay, 0)

        def drain(i, _):
            pltpu.make_async_copy(
                binned_hbm.at[pl.ds(0, _WINW)],
                seg_st.at[pl.ds(i * _WINW, _WINW)], sem).wait()
            return 0

        lax.fori_loop(0, _NW, drain, 0)

        def seg(i, _):
            p0 = par_st[pl.ds(i, _L)][0]
            ln = par_st[pl.ds(2 * _L + i, _L)][0]
            shp = p0 & 7
            sbase = i * _WINW

            def vec(k, _):
                wpos = k * _L + lanes
                vi = plsc.load_gather(seg_st, [sbase + wpos * 2])
                vv = plsc.bitcast(
                    plsc.load_gather(seg_st, [sbase + wpos * 2 + 1]), jnp.float32)
                m0 = (wpos >= shp) & (wpos < shp + _CH) & (wpos - shp < ln)
                lo = vi & (_V - 1)

                def retry(m):
                    g = plsc.load_gather(acc, [lo], mask=m)
                    plsc.store_scatter(acc, [lo], jnp.maximum(g, vv), mask=m)
                    g2 = plsc.load_gather(acc, [lo], mask=m)
                    return m & (g2 < vv)

                lax.while_loop(
                    lambda m: plsc.all_reduce_population_count(m)[0] != 0,
                    retry, m0)
                return 0

            nkv = lax.shift_right_logical(
                shp + jnp.minimum(ln, _CH) + _L - 1, 4)
            lax.fori_loop(0, nkv, vec, 0)

            nch = lax.shift_right_logical(ln + _CH - 1, _CH.bit_length() - 1)

            def chunk(j, _):
                pj = p0 + j * _CH
                a0 = pl.multiple_of((pj * 2) & ~15, 8)
                shp2 = pj & 7
                pltpu.sync_copy(binned_hbm.at[pl.ds(a0, _WINW)], pair_st)

                def vec2(k, _):
                    wpos = k * _L + lanes
                    vi = plsc.load_gather(pair_st, [wpos * 2])
                    vv = plsc.bitcast(
                        plsc.load_gather(pair_st, [wpos * 2 + 1]), jnp.float32)
                    q = wpos - shp2 + j * _CH
                    m0 = (wpos >= shp2) & (wpos < shp2 + _CH) & (q < ln)
                    lo = vi & (_V - 1)

                    def retry(m):
                        g = plsc.load_gather(acc, [lo], mask=m)
                        plsc.store_scatter(acc, [lo], jnp.maximum(g, vv), mask=m)
                        g2 = plsc.load_gather(acc, [lo], mask=m)
                        return m & (g2 < vv)

                    lax.while_loop(
                        lambda m: plsc.all_reduce_population_count(m)[0] != 0,
                        retry, m0)
                    return 0

                nkv2 = lax.shift_right_logical(
                    shp2 + jnp.minimum(ln - j * _CH, _CH) + _L - 1, 4)
                lax.fori_loop(0, nkv2, vec2, 0)
                return 0

            lax.fori_loop(1, nch, chunk, 0)
            return 0

        lax.fori_loop(0, _NW, seg, 0)
        pltpu.sync_copy(acc, new_hbm.at[pl.ds(gbase, _V)])
        return pre0 + ln0, pre1 + ln1

    lax.fori_loop(0, _BUK_PER_W, bucket_body, (pre0, pre1))


def _thr_body(x_ref, o_ref):
    o_ref[...] = x_ref[...] > _OCC_THRE


_ROWS = _GRID // 128


def _phase3(new2d):
    return pl.pallas_call(
        _thr_body,
        out_shape=jax.ShapeDtypeStruct((_ROWS, 128), jnp.bool_),
        grid=(32,),
        in_specs=[pl.BlockSpec((_ROWS // 32, 128), lambda i: (i, 0))],
        out_specs=pl.BlockSpec((_ROWS // 32, 128), lambda i: (i, 0)),
    )(new2d)


def kernel(occ_val_grid, pts, bidx, val):
    pts_planar = pts.T.reshape(-1)
    occ_flat = occ_val_grid.reshape(-1)
    binned, totals = _phase1(pts_planar, bidx, val)
    new_flat = _phase2(occ_flat, binned, totals)
    occ_bool = _phase3(new_flat.reshape(_ROWS, 128))
    shape = occ_val_grid.shape
    return new_flat.reshape(shape), occ_bool.reshape(shape)
